# baseline (speedup 1.0000x reference)
import functools

import jax
import jax.numpy as jnp
from jax import lax
from jax.experimental import pallas as pl
from jax.experimental.pallas import tpu as pltpu
from jax.experimental.pallas import tpu_sc as plsc

D_MODEL = 768
NUM_EXPERTS = 8
TOP_K = 2
MASK_W = TOP_K * NUM_EXPERTS


def _logits_body(x_ref, w_ref, out_ref):
    out_ref[...] = lax.dot_general(
        x_ref[...], w_ref[...],
        dimension_numbers=(((1,), (1,)), ((), ())),
        preferred_element_type=jnp.float32)


def _compute_logits(x, W):
    n = x.shape[0]
    tn = 1024
    return pl.pallas_call(
        _logits_body,
        grid=(n // tn,),
        in_specs=[pl.BlockSpec((tn, D_MODEL), lambda i: (i, 0)),
                  pl.BlockSpec((NUM_EXPERTS, D_MODEL), lambda i: (0, 0))],
        out_specs=pl.BlockSpec((tn, NUM_EXPERTS), lambda i: (i, 0)),
        out_shape=jax.ShapeDtypeStruct((n, NUM_EXPERTS), jnp.float32),
    )(x, W)


@functools.lru_cache(maxsize=None)
def _make_router(n):
    info = plsc.get_sparse_core_info()
    nc, ns, lanes = info.num_cores, info.num_subcores, info.num_lanes
    nw = nc * ns
    tpw = n // nw
    mesh = plsc.VectorSubcoreMesh(core_axis_name="c", subcore_axis_name="s")

    @functools.partial(
        pl.kernel, mesh=mesh,
        compiler_params=pltpu.CompilerParams(needs_layout_passes=False),
        out_type=[
            jax.ShapeDtypeStruct((n * TOP_K,), jnp.float32),
            jax.ShapeDtypeStruct((n * TOP_K,), jnp.int32),
            jax.ShapeDtypeStruct((n * MASK_W,), jnp.float32),
        ],
        scratch_types=[
            pltpu.VMEM((tpw * NUM_EXPERTS,), jnp.float32),
            pltpu.VMEM((tpw * TOP_K,), jnp.float32),
            pltpu.VMEM((tpw * TOP_K,), jnp.int32),
            pltpu.VMEM((tpw * MASK_W,), jnp.float32),
        ],
    )
    def router(logits_hbm, probs_hbm, idx_hbm, mask_hbm,
               logits_v, probs_v, idx_v, mask_v):
        wid = lax.axis_index("s") * nc + lax.axis_index("c")
        base = wid * tpw
        pltpu.sync_copy(
            logits_hbm.at[pl.ds(base * NUM_EXPERTS, tpw * NUM_EXPERTS)],
            logits_v)

        def chunk(i, carry):
            rows = i * lanes + lax.iota(jnp.int32, lanes)
            r8 = rows * NUM_EXPERTS
            e = [plsc.load_gather(logits_v, [r8 + j])
                 for j in range(NUM_EXPERTS)]
            m1 = e[0]
            i1 = jnp.zeros((lanes,), jnp.int32)
            for j in range(1, NUM_EXPERTS):
                gt = e[j] > m1
                m1 = jnp.where(gt, e[j], m1)
                i1 = jnp.where(gt, j, i1)
            m2 = jnp.full((lanes,), -3e38, jnp.float32)
            i2 = jnp.zeros((lanes,), jnp.int32)
            for j in range(NUM_EXPERTS):
                gt = (e[j] > m2) & (i1 != j)
                m2 = jnp.where(gt, e[j], m2)
                i2 = jnp.where(gt, j, i2)
            d = jnp.exp(m2 - m1)
            p1 = 1.0 / (1.0 + d)
            p2 = d * p1
            r2 = rows * TOP_K
            plsc.store_scatter(probs_v, [r2], p1)
            plsc.store_scatter(probs_v, [r2 + 1], p2)
            plsc.store_scatter(idx_v, [r2], i1)
            plsc.store_scatter(idx_v, [r2 + 1], i2)
            r16 = rows * MASK_W
            for k in range(NUM_EXPERTS):
                plsc.store_scatter(
                    mask_v, [r16 + k], jnp.where(i1 == k, 1.0, 0.0))
                plsc.store_scatter(
                    mask_v, [r16 + NUM_EXPERTS + k],
                    jnp.where(i2 == k, 1.0, 0.0))
            return carry

        lax.fori_loop(0, tpw // lanes, chunk, 0)
        pltpu.sync_copy(probs_v, probs_hbm.at[pl.ds(base * TOP_K, tpw * TOP_K)])
        pltpu.sync_copy(idx_v, idx_hbm.at[pl.ds(base * TOP_K, tpw * TOP_K)])
        pltpu.sync_copy(mask_v, mask_hbm.at[pl.ds(base * MASK_W, tpw * MASK_W)])

    return router


def kernel(x, W):
    n = x.shape[0]
    logits = _compute_logits(x, W)
    probs, idx, mask = _make_router(n)(logits.reshape(n * NUM_EXPERTS))
    return (probs.reshape(n, TOP_K), idx.reshape(n, TOP_K),
            mask.reshape(n, TOP_K, NUM_EXPERTS))


# speedup vs baseline: 1.0034x; 1.0034x over previous
import functools

import jax
import jax.numpy as jnp
from jax import lax
from jax.experimental import pallas as pl
from jax.experimental.pallas import tpu as pltpu
from jax.experimental.pallas import tpu_sc as plsc

D_MODEL = 768
NUM_EXPERTS = 8
TOP_K = 2


def _logits_body(x_ref, w_ref, out_ref):
    out_ref[...] = lax.dot_general(
        x_ref[...], w_ref[...],
        dimension_numbers=(((1,), (1,)), ((), ())),
        preferred_element_type=jnp.float32)


def _compute_logits(x, W):
    n = x.shape[0]
    tn = 1024
    return pl.pallas_call(
        _logits_body,
        grid=(n // tn,),
        in_specs=[pl.BlockSpec((tn, D_MODEL), lambda i: (i, 0)),
                  pl.BlockSpec((NUM_EXPERTS, D_MODEL), lambda i: (0, 0))],
        out_specs=pl.BlockSpec((tn, NUM_EXPERTS), lambda i: (i, 0)),
        out_shape=jax.ShapeDtypeStruct((n, NUM_EXPERTS), jnp.float32),
    )(x, W)


@functools.lru_cache(maxsize=None)
def _make_router(n):
    info = plsc.get_sparse_core_info()
    nc, ns, lanes = info.num_cores, info.num_subcores, info.num_lanes
    nw = nc * ns
    tpw = n // nw
    mesh = plsc.VectorSubcoreMesh(core_axis_name="c", subcore_axis_name="s")

    @functools.partial(
        pl.kernel, mesh=mesh,
        compiler_params=pltpu.CompilerParams(
            needs_layout_passes=False, use_tc_tiling_on_sc=False),
        out_type=[
            jax.ShapeDtypeStruct((n, TOP_K), jnp.float32),
            jax.ShapeDtypeStruct((n, TOP_K), jnp.int32),
            jax.ShapeDtypeStruct((n, TOP_K, NUM_EXPERTS), jnp.float32),
        ],
        scratch_types=[
            pltpu.VMEM((tpw, NUM_EXPERTS), jnp.float32),
            pltpu.VMEM((tpw, TOP_K), jnp.float32),
            pltpu.VMEM((tpw, TOP_K), jnp.int32),
            pltpu.VMEM((tpw, TOP_K, NUM_EXPERTS), jnp.float32),
        ],
    )
    def router(logits_hbm, probs_hbm, idx_hbm, mask_hbm,
               logits_v, probs_v, idx_v, mask_v):
        wid = lax.axis_index("s") * nc + lax.axis_index("c")
        base = wid * tpw
        pltpu.sync_copy(logits_hbm.at[pl.ds(base, tpw)], logits_v)

        zero = jnp.zeros((lanes,), jnp.int32)
        one = jnp.ones((lanes,), jnp.int32)

        def chunk(i, carry):
            rows = i * lanes + lax.iota(jnp.int32, lanes)
            e = [plsc.load_gather(
                    logits_v, [rows, jnp.full((lanes,), j, jnp.int32)])
                 for j in range(NUM_EXPERTS)]
            m1 = e[0]
            i1 = zero
            for j in range(1, NUM_EXPERTS):
                gt = e[j] > m1
                m1 = jnp.where(gt, e[j], m1)
                i1 = jnp.where(gt, j, i1)
            m2 = jnp.full((lanes,), -3e38, jnp.float32)
            i2 = zero
            for j in range(NUM_EXPERTS):
                gt = (e[j] > m2) & (i1 != j)
                m2 = jnp.where(gt, e[j], m2)
                i2 = jnp.where(gt, j, i2)
            d = jnp.exp(m2 - m1)
            p1 = 1.0 / (1.0 + d)
            p2 = d * p1
            plsc.store_scatter(probs_v, [rows, zero], p1)
            plsc.store_scatter(probs_v, [rows, one], p2)
            plsc.store_scatter(idx_v, [rows, zero], i1)
            plsc.store_scatter(idx_v, [rows, one], i2)
            for k in range(NUM_EXPERTS):
                ck = jnp.full((lanes,), k, jnp.int32)
                plsc.store_scatter(
                    mask_v, [rows, zero, ck], jnp.where(i1 == k, 1.0, 0.0))
                plsc.store_scatter(
                    mask_v, [rows, one, ck], jnp.where(i2 == k, 1.0, 0.0))
            return carry

        lax.fori_loop(0, tpw // lanes, chunk, 0)
        pltpu.sync_copy(probs_v, probs_hbm.at[pl.ds(base, tpw)])
        pltpu.sync_copy(idx_v, idx_hbm.at[pl.ds(base, tpw)])
        pltpu.sync_copy(mask_v, mask_hbm.at[pl.ds(base, tpw)])

    return router


def kernel(x, W):
    n = x.shape[0]
    logits = _compute_logits(x, W)
    probs, idx, mask = _make_router(n)(logits)
    return probs, idx, mask


# speedup vs baseline: 3.0781x; 3.0677x over previous
import functools

import jax
import jax.numpy as jnp
from jax import lax
from jax.experimental import pallas as pl
from jax.experimental.pallas import tpu as pltpu
from jax.experimental.pallas import tpu_sc as plsc

D_MODEL = 768
NUM_EXPERTS = 8
TOP_K = 2
MASK_W = TOP_K * NUM_EXPERTS


def _logits_body(x_ref, w_ref, out_ref):
    out_ref[...] = lax.dot_general(
        w_ref[...], x_ref[...],
        dimension_numbers=(((1,), (1,)), ((), ())),
        preferred_element_type=jnp.float32)


def _compute_logits_t(x, W):
    n = x.shape[0]
    tn = 1024
    return pl.pallas_call(
        _logits_body,
        grid=(n // tn,),
        in_specs=[pl.BlockSpec((tn, D_MODEL), lambda i: (i, 0)),
                  pl.BlockSpec((NUM_EXPERTS, D_MODEL), lambda i: (0, 0))],
        out_specs=pl.BlockSpec((NUM_EXPERTS, tn), lambda i: (0, i)),
        out_shape=jax.ShapeDtypeStruct((NUM_EXPERTS, n), jnp.float32),
    )(x, W)


@functools.lru_cache(maxsize=None)
def _make_router(n):
    info = plsc.get_sparse_core_info()
    nc, ns, lanes = info.num_cores, info.num_subcores, info.num_lanes
    nw = nc * ns
    tpw = n // nw
    mesh = plsc.VectorSubcoreMesh(core_axis_name="c", subcore_axis_name="s")

    @functools.partial(
        pl.kernel, mesh=mesh,
        compiler_params=pltpu.CompilerParams(
            needs_layout_passes=False, use_tc_tiling_on_sc=False),
        out_type=[
            jax.ShapeDtypeStruct((TOP_K * n,), jnp.float32),
            jax.ShapeDtypeStruct((TOP_K * n,), jnp.int32),
            jax.ShapeDtypeStruct((MASK_W * n,), jnp.float32),
        ],
        scratch_types=(
            [pltpu.VMEM((tpw,), jnp.float32) for _ in range(NUM_EXPERTS)]
            + [pltpu.VMEM((tpw,), jnp.float32) for _ in range(TOP_K)]
            + [pltpu.VMEM((tpw,), jnp.int32) for _ in range(TOP_K)]
            + [pltpu.VMEM((tpw,), jnp.float32) for _ in range(MASK_W)]
        ),
    )
    def router(logits_hbm, probs_hbm, idx_hbm, mask_hbm, *scratch):
        e_v = scratch[0:NUM_EXPERTS]
        p_v = scratch[NUM_EXPERTS:NUM_EXPERTS + TOP_K]
        ix_v = scratch[NUM_EXPERTS + TOP_K:NUM_EXPERTS + 2 * TOP_K]
        m_v = scratch[NUM_EXPERTS + 2 * TOP_K:]
        wid = lax.axis_index("s") * nc + lax.axis_index("c")
        base = wid * tpw
        for j in range(NUM_EXPERTS):
            pltpu.sync_copy(logits_hbm.at[pl.ds(j * n + base, tpw)], e_v[j])

        def chunk(i, carry):
            sl = pl.ds(i * lanes, lanes)
            e = [e_v[j][sl] for j in range(NUM_EXPERTS)]
            m1 = e[0]
            i1 = jnp.zeros((lanes,), jnp.int32)
            for j in range(1, NUM_EXPERTS):
                gt = e[j] > m1
                m1 = jnp.where(gt, e[j], m1)
                i1 = jnp.where(gt, j, i1)
            m2 = jnp.full((lanes,), -3e38, jnp.float32)
            i2 = jnp.zeros((lanes,), jnp.int32)
            for j in range(NUM_EXPERTS):
                gt = (e[j] > m2) & (i1 != j)
                m2 = jnp.where(gt, e[j], m2)
                i2 = jnp.where(gt, j, i2)
            d = jnp.exp(m2 - m1)
            p1 = 1.0 / (1.0 + d)
            p2 = d * p1
            p_v[0][sl] = p1
            p_v[1][sl] = p2
            ix_v[0][sl] = i1
            ix_v[1][sl] = i2
            for k in range(NUM_EXPERTS):
                m_v[k][sl] = jnp.where(i1 == k, 1.0, 0.0)
                m_v[NUM_EXPERTS + k][sl] = jnp.where(i2 == k, 1.0, 0.0)
            return carry

        lax.fori_loop(0, tpw // lanes, chunk, 0)
        for r in range(TOP_K):
            pltpu.sync_copy(p_v[r], probs_hbm.at[pl.ds(r * n + base, tpw)])
            pltpu.sync_copy(ix_v[r], idx_hbm.at[pl.ds(r * n + base, tpw)])
        for k in range(MASK_W):
            pltpu.sync_copy(m_v[k], mask_hbm.at[pl.ds(k * n + base, tpw)])

    return router


def kernel(x, W):
    n = x.shape[0]
    logits_t = _compute_logits_t(x, W)
    probs_t, idx_t, mask_t = _make_router(n)(logits_t.reshape(-1))
    probs = probs_t.reshape(TOP_K, n).T
    idx = idx_t.reshape(TOP_K, n).T
    mask = mask_t.reshape(TOP_K, NUM_EXPERTS, n).transpose(2, 0, 1)
    return probs, idx, mask


# speedup vs baseline: 3.5593x; 1.1564x over previous
import functools

import jax
import jax.numpy as jnp
from jax import lax
from jax.experimental import pallas as pl
from jax.experimental.pallas import tpu as pltpu
from jax.experimental.pallas import tpu_sc as plsc

D_MODEL = 768
NUM_EXPERTS = 8
TOP_K = 2
MASK_W = TOP_K * NUM_EXPERTS


def _logits_body(x_ref, w_ref, out_ref):
    out_ref[...] = lax.dot_general(
        w_ref[...], x_ref[...],
        dimension_numbers=(((1,), (1,)), ((), ())),
        preferred_element_type=jnp.float32)


def _compute_logits_t(x, W):
    n = x.shape[0]
    tn = 4096
    return pl.pallas_call(
        _logits_body,
        grid=(n // tn,),
        in_specs=[pl.BlockSpec((tn, D_MODEL), lambda i: (i, 0)),
                  pl.BlockSpec((NUM_EXPERTS, D_MODEL), lambda i: (0, 0))],
        out_specs=pl.BlockSpec((NUM_EXPERTS, tn), lambda i: (0, i)),
        out_shape=jax.ShapeDtypeStruct((NUM_EXPERTS, n), jnp.float32),
    )(x, W)


@functools.lru_cache(maxsize=None)
def _make_router(n):
    info = plsc.get_sparse_core_info()
    nc, ns, lanes = info.num_cores, info.num_subcores, info.num_lanes
    nw = nc * ns
    tpw = n // nw
    mesh = plsc.VectorSubcoreMesh(core_axis_name="c", subcore_axis_name="s")

    @functools.partial(
        pl.kernel, mesh=mesh,
        compiler_params=pltpu.CompilerParams(
            needs_layout_passes=False, use_tc_tiling_on_sc=False),
        out_type=[
            jax.ShapeDtypeStruct((TOP_K * n,), jnp.float32),
            jax.ShapeDtypeStruct((TOP_K * n,), jnp.int32),
            jax.ShapeDtypeStruct((MASK_W * n,), jnp.float32),
        ],
        scratch_types=(
            [pltpu.VMEM((tpw,), jnp.float32) for _ in range(NUM_EXPERTS)]
            + [pltpu.VMEM((tpw,), jnp.float32) for _ in range(TOP_K)]
            + [pltpu.VMEM((tpw,), jnp.int32) for _ in range(TOP_K)]
            + [pltpu.VMEM((tpw,), jnp.float32) for _ in range(MASK_W)]
        ),
    )
    def router(logits_hbm, probs_hbm, idx_hbm, mask_hbm, *scratch):
        e_v = scratch[0:NUM_EXPERTS]
        p_v = scratch[NUM_EXPERTS:NUM_EXPERTS + TOP_K]
        ix_v = scratch[NUM_EXPERTS + TOP_K:NUM_EXPERTS + 2 * TOP_K]
        m_v = scratch[NUM_EXPERTS + 2 * TOP_K:]
        wid = lax.axis_index("s") * nc + lax.axis_index("c")
        base = wid * tpw
        for j in range(NUM_EXPERTS):
            pltpu.sync_copy(logits_hbm.at[pl.ds(j * n + base, tpw)], e_v[j])

        def chunk(i, carry):
            sl = pl.ds(i * lanes, lanes)
            e = [e_v[j][sl] for j in range(NUM_EXPERTS)]
            m1 = e[0]
            i1 = jnp.zeros((lanes,), jnp.int32)
            for j in range(1, NUM_EXPERTS):
                gt = e[j] > m1
                m1 = jnp.where(gt, e[j], m1)
                i1 = jnp.where(gt, j, i1)
            m2 = jnp.full((lanes,), -3e38, jnp.float32)
            i2 = jnp.zeros((lanes,), jnp.int32)
            for j in range(NUM_EXPERTS):
                gt = (e[j] > m2) & (i1 != j)
                m2 = jnp.where(gt, e[j], m2)
                i2 = jnp.where(gt, j, i2)
            d = jnp.exp(m2 - m1)
            p1 = 1.0 / (1.0 + d)
            p2 = d * p1
            p_v[0][sl] = p1
            p_v[1][sl] = p2
            ix_v[0][sl] = i1
            ix_v[1][sl] = i2
            for k in range(NUM_EXPERTS):
                m_v[k][sl] = jnp.where(i1 == k, 1.0, 0.0)
                m_v[NUM_EXPERTS + k][sl] = jnp.where(i2 == k, 1.0, 0.0)
            return carry

        lax.fori_loop(0, tpw // lanes, chunk, 0)
        for r in range(TOP_K):
            pltpu.sync_copy(p_v[r], probs_hbm.at[pl.ds(r * n + base, tpw)])
            pltpu.sync_copy(ix_v[r], idx_hbm.at[pl.ds(r * n + base, tpw)])
        for k in range(MASK_W):
            pltpu.sync_copy(m_v[k], mask_hbm.at[pl.ds(k * n + base, tpw)])

    return router


def kernel(x, W):
    n = x.shape[0]
    logits_t = _compute_logits_t(x, W)
    probs_t, idx_t, mask_t = _make_router(n)(logits_t.reshape(-1))
    probs = probs_t.reshape(TOP_K, n).T
    idx = idx_t.reshape(TOP_K, n).T
    mask = mask_t.reshape(TOP_K, NUM_EXPERTS, n).transpose(2, 0, 1)
    return probs, idx, mask
